# initial kernel scaffold (unmeasured)
import functools

import jax
import jax.numpy as jnp
from jax import lax
from jax.experimental import pallas as pl
from jax.experimental.pallas import tpu as pltpu

N_DEV = 32


def kernel(x, w_mat):
    m_glob, k_loc = x.shape
    k2, n = w_mat.shape
    assert k_loc == k2
    blk = m_glob // N_DEV

    def body(x_ref, w_ref, out_ref, amax_ref,
             slots, send_sems, recv_sems,
             bf_send, bf_recv, bf_send_sems, bf_recv_sems):
        my = lax.axis_index("i")
        left = lax.rem(my - 1 + N_DEV, N_DEV)
        right = lax.rem(my + 1, N_DEV)

        barrier_sem = pltpu.get_barrier_semaphore()
        for nbr in (left, right):
            pl.semaphore_signal(
                barrier_sem, inc=1,
                device_id=(nbr,), device_id_type=pl.DeviceIdType.MESH,
            )
        pl.semaphore_wait(barrier_sem, 2)

        send_rdmas = []
        for h in range(N_DEV):
            o = lax.rem(my - 1 - h + 2 * N_DEV, N_DEV)
            p = jnp.dot(
                x_ref[pl.ds(o * blk, blk), :], w_ref[...],
                preferred_element_type=jnp.float32,
            )
            if h == 0:
                slots[h] = p
            else:
                recv = pltpu.make_async_remote_copy(
                    src_ref=slots.at[h], dst_ref=slots.at[h],
                    send_sem=send_sems.at[h], recv_sem=recv_sems.at[h],
                    device_id=(left,), device_id_type=pl.DeviceIdType.MESH,
                )
                recv.wait_recv()
                if h < N_DEV - 1:
                    slots[h] = slots[h] + p
                else:
                    out_ref[...] = slots[h] + p
            if h < N_DEV - 1:
                rdma = pltpu.make_async_remote_copy(
                    src_ref=slots.at[h], dst_ref=slots.at[h + 1],
                    send_sem=send_sems.at[h], recv_sem=recv_sems.at[h + 1],
                    device_id=(right,), device_id_type=pl.DeviceIdType.MESH,
                )
                rdma.start()
                send_rdmas.append(rdma)

        amax = jnp.max(jnp.abs(out_ref[...]))
        for k in range(5):
            partner = my ^ (1 << k)
            bf_send[k] = jnp.broadcast_to(amax, (8, 128))
            ex = pltpu.make_async_remote_copy(
                src_ref=bf_send.at[k], dst_ref=bf_recv.at[k],
                send_sem=bf_send_sems.at[k], recv_sem=bf_recv_sems.at[k],
                device_id=(partner,), device_id_type=pl.DeviceIdType.MESH,
            )
            ex.start()
            ex.wait()
            amax = jnp.maximum(amax, bf_recv[k][0, 0])
        amax_ref[...] = jnp.broadcast_to(amax, (8, 128))

        for rdma in send_rdmas:
            rdma.wait_send()

        @functools.partial(
            pl.run_scoped, second_barrier=pltpu.SemaphoreType.REGULAR
        )
        def _(second_barrier):
            for nbr in (left, right):
                pl.semaphore_signal(
                    second_barrier, inc=1,
                    device_id=(nbr,), device_id_type=pl.DeviceIdType.MESH,
                )
            pl.semaphore_wait(second_barrier, 2)

    y, amax = pl.pallas_call(
        body,
        out_shape=[
            jax.ShapeDtypeStruct((blk, n), jnp.float32),
            jax.ShapeDtypeStruct((8, 128), jnp.float32),
        ],
        in_specs=[
            pl.BlockSpec(memory_space=pltpu.VMEM),
            pl.BlockSpec(memory_space=pltpu.VMEM),
        ],
        out_specs=[
            pl.BlockSpec(memory_space=pltpu.VMEM),
            pl.BlockSpec(memory_space=pltpu.VMEM),
        ],
        scratch_shapes=[
            pltpu.VMEM((N_DEV, blk, n), jnp.float32),
            pltpu.SemaphoreType.DMA((N_DEV,)),
            pltpu.SemaphoreType.DMA((N_DEV,)),
            pltpu.VMEM((5, 8, 128), jnp.float32),
            pltpu.VMEM((5, 8, 128), jnp.float32),
            pltpu.SemaphoreType.DMA((5,)),
            pltpu.SemaphoreType.DMA((5,)),
        ],
        compiler_params=pltpu.CompilerParams(collective_id=0),
    )(x, w_mat)

    scale = amax[0, 0] / 448.0
    q = jnp.clip(y / scale, -448.0, 448.0)
    q = q.astype(jnp.float8_e4m3fn).astype(jnp.float32)
    return q * scale


# baseline (device time: 434022 ns/iter reference)
import functools

import jax
import jax.numpy as jnp
from jax import lax
from jax.experimental import pallas as pl
from jax.experimental.pallas import tpu as pltpu

N_DEV = 32


def kernel(x, w_mat):
    m_glob, k_loc = x.shape
    k2, n = w_mat.shape
    assert k_loc == k2
    blk = m_glob // N_DEV

    def body(x_ref, w_ref, out_ref, amax_ref,
             slots, send_sems, recv_sems,
             bf_send, bf_recv, bf_send_sems, bf_recv_sems):
        my = lax.axis_index("i")
        left = lax.rem(my - 1 + N_DEV, N_DEV)
        right = lax.rem(my + 1, N_DEV)

        barrier_sem = pltpu.get_barrier_semaphore()
        for nbr in (left, right):
            pl.semaphore_signal(
                barrier_sem, inc=1,
                device_id=(nbr,), device_id_type=pl.DeviceIdType.MESH,
            )
        pl.semaphore_wait(barrier_sem, 2)

        send_rdmas = []
        for h in range(N_DEV):
            o = lax.rem(my - 1 - h + 2 * N_DEV, N_DEV)
            p = jnp.dot(
                x_ref[pl.ds(o * blk, blk), :], w_ref[...],
                preferred_element_type=jnp.float32,
                precision=lax.Precision.HIGHEST,
            )
            if h == 0:
                slots[h] = p
            else:
                recv = pltpu.make_async_remote_copy(
                    src_ref=slots.at[h], dst_ref=slots.at[h],
                    send_sem=send_sems.at[h], recv_sem=recv_sems.at[h],
                    device_id=(left,), device_id_type=pl.DeviceIdType.MESH,
                )
                recv.wait_recv()
                if h < N_DEV - 1:
                    slots[h] = slots[h] + p
                else:
                    out_ref[...] = slots[h] + p
            if h < N_DEV - 1:
                rdma = pltpu.make_async_remote_copy(
                    src_ref=slots.at[h], dst_ref=slots.at[h + 1],
                    send_sem=send_sems.at[h], recv_sem=recv_sems.at[h + 1],
                    device_id=(right,), device_id_type=pl.DeviceIdType.MESH,
                )
                rdma.start()
                send_rdmas.append(rdma)

        amax = jnp.max(jnp.abs(out_ref[...]))
        for k in range(5):
            partner = my ^ (1 << k)
            bf_send[k] = jnp.broadcast_to(amax, (8, 128))
            ex = pltpu.make_async_remote_copy(
                src_ref=bf_send.at[k], dst_ref=bf_recv.at[k],
                send_sem=bf_send_sems.at[k], recv_sem=bf_recv_sems.at[k],
                device_id=(partner,), device_id_type=pl.DeviceIdType.MESH,
            )
            ex.start()
            ex.wait()
            amax = jnp.maximum(amax, bf_recv[k][0, 0])
        amax_ref[...] = jnp.broadcast_to(amax, (8, 128))

        scale = amax / 448.0
        v = jnp.clip(out_ref[...] / scale, -448.0, 448.0)
        q = v.astype(jnp.float8_e4m3fn).astype(jnp.float32)
        out_ref[...] = q * scale

        for rdma in send_rdmas:
            rdma.wait_send()

        @functools.partial(
            pl.run_scoped, second_barrier=pltpu.SemaphoreType.REGULAR
        )
        def _(second_barrier):
            for nbr in (left, right):
                pl.semaphore_signal(
                    second_barrier, inc=1,
                    device_id=(nbr,), device_id_type=pl.DeviceIdType.MESH,
                )
            pl.semaphore_wait(second_barrier, 2)

    y, amax = pl.pallas_call(
        body,
        out_shape=[
            jax.ShapeDtypeStruct((blk, n), jnp.float32),
            jax.ShapeDtypeStruct((8, 128), jnp.float32),
        ],
        in_specs=[
            pl.BlockSpec(memory_space=pltpu.VMEM),
            pl.BlockSpec(memory_space=pltpu.VMEM),
        ],
        out_specs=[
            pl.BlockSpec(memory_space=pltpu.VMEM),
            pl.BlockSpec(memory_space=pltpu.VMEM),
        ],
        scratch_shapes=[
            pltpu.VMEM((N_DEV, blk, n), jnp.float32),
            pltpu.SemaphoreType.DMA((N_DEV,)),
            pltpu.SemaphoreType.DMA((N_DEV,)),
            pltpu.VMEM((5, 8, 128), jnp.float32),
            pltpu.VMEM((5, 8, 128), jnp.float32),
            pltpu.SemaphoreType.DMA((5,)),
            pltpu.SemaphoreType.DMA((5,)),
        ],
        compiler_params=pltpu.CompilerParams(
            collective_id=0,
            vmem_limit_bytes=100 * 1024 * 1024,
        ),
    )(x, w_mat)

    del amax
    return y


# device time: 407413 ns/iter; 1.0653x vs baseline; 1.0653x over previous
import functools

import jax
import jax.numpy as jnp
from jax import lax
from jax.experimental import pallas as pl
from jax.experimental.pallas import tpu as pltpu

N_DEV = 32


def kernel(x, w_mat):
    m_glob, k_loc = x.shape
    k2, n = w_mat.shape
    assert k_loc == k2
    blk = m_glob // N_DEV
    nh = n // 2

    def body(x_ref, w_ref, out_ref, amax_ref,
             slots_r, slots_l,
             send_sems_r, recv_sems_r, send_sems_l, recv_sems_l,
             bf_send, bf_recv, bf_send_sems, bf_recv_sems):
        my = lax.axis_index("i")
        left = lax.rem(my - 1 + N_DEV, N_DEV)
        right = lax.rem(my + 1, N_DEV)

        barrier_sem = pltpu.get_barrier_semaphore()
        for nbr in (left, right):
            pl.semaphore_signal(
                barrier_sem, inc=1,
                device_id=(nbr,), device_id_type=pl.DeviceIdType.MESH,
            )
        pl.semaphore_wait(barrier_sem, 2)

        send_rdmas = []

        def hop_dir(h, slots, send_sems, recv_sems, o, cols, dst):
            p = jnp.dot(
                x_ref[pl.ds(o * blk, blk), :], w_ref[:, cols],
                preferred_element_type=jnp.float32,
                precision=lax.Precision.HIGHEST,
            )
            last = h == N_DEV - 1
            if h == 0:
                slots[h] = p
            else:
                recv = pltpu.make_async_remote_copy(
                    src_ref=slots.at[h], dst_ref=slots.at[h],
                    send_sem=send_sems.at[h], recv_sem=recv_sems.at[h],
                    device_id=(dst,), device_id_type=pl.DeviceIdType.MESH,
                )
                recv.wait_recv()
                if last:
                    return slots[h] + p
                slots[h] = slots[h] + p
            rdma = pltpu.make_async_remote_copy(
                src_ref=slots.at[h], dst_ref=slots.at[h + 1],
                send_sem=send_sems.at[h], recv_sem=recv_sems.at[h + 1],
                device_id=(dst,), device_id_type=pl.DeviceIdType.MESH,
            )
            rdma.start()
            send_rdmas.append(rdma)
            return None

        for h in range(N_DEV):
            o_r = lax.rem(my - 1 - h + 2 * N_DEV, N_DEV)
            o_l = lax.rem(my + 1 + h, N_DEV)
            acc_r = hop_dir(h, slots_r, send_sems_r, recv_sems_r,
                            o_r, pl.ds(0, nh), right)
            acc_l = hop_dir(h, slots_l, send_sems_l, recv_sems_l,
                            o_l, pl.ds(nh, nh), left)
            if h == N_DEV - 1:
                out_ref[:, pl.ds(0, nh)] = acc_r
                out_ref[:, pl.ds(nh, nh)] = acc_l

        amax = jnp.max(jnp.abs(out_ref[...]))
        for k in range(5):
            partner = my ^ (1 << k)
            bf_send[k] = jnp.broadcast_to(amax, (8, 128))
            ex = pltpu.make_async_remote_copy(
                src_ref=bf_send.at[k], dst_ref=bf_recv.at[k],
                send_sem=bf_send_sems.at[k], recv_sem=bf_recv_sems.at[k],
                device_id=(partner,), device_id_type=pl.DeviceIdType.MESH,
            )
            ex.start()
            ex.wait()
            amax = jnp.maximum(amax, bf_recv[k][0, 0])
        amax_ref[...] = jnp.broadcast_to(amax, (8, 128))

        scale = amax / 448.0
        v = jnp.clip(out_ref[...] / scale, -448.0, 448.0)
        q = v.astype(jnp.float8_e4m3fn).astype(jnp.float32)
        out_ref[...] = q * scale

        for rdma in send_rdmas:
            rdma.wait_send()

        @functools.partial(
            pl.run_scoped, second_barrier=pltpu.SemaphoreType.REGULAR
        )
        def _(second_barrier):
            for nbr in (left, right):
                pl.semaphore_signal(
                    second_barrier, inc=1,
                    device_id=(nbr,), device_id_type=pl.DeviceIdType.MESH,
                )
            pl.semaphore_wait(second_barrier, 2)

    y, amax = pl.pallas_call(
        body,
        out_shape=[
            jax.ShapeDtypeStruct((blk, n), jnp.float32),
            jax.ShapeDtypeStruct((8, 128), jnp.float32),
        ],
        in_specs=[
            pl.BlockSpec(memory_space=pltpu.VMEM),
            pl.BlockSpec(memory_space=pltpu.VMEM),
        ],
        out_specs=[
            pl.BlockSpec(memory_space=pltpu.VMEM),
            pl.BlockSpec(memory_space=pltpu.VMEM),
        ],
        scratch_shapes=[
            pltpu.VMEM((N_DEV, blk, nh), jnp.float32),
            pltpu.VMEM((N_DEV, blk, nh), jnp.float32),
            pltpu.SemaphoreType.DMA((N_DEV,)),
            pltpu.SemaphoreType.DMA((N_DEV,)),
            pltpu.SemaphoreType.DMA((N_DEV,)),
            pltpu.SemaphoreType.DMA((N_DEV,)),
            pltpu.VMEM((5, 8, 128), jnp.float32),
            pltpu.VMEM((5, 8, 128), jnp.float32),
            pltpu.SemaphoreType.DMA((5,)),
            pltpu.SemaphoreType.DMA((5,)),
        ],
        compiler_params=pltpu.CompilerParams(
            collective_id=0,
            vmem_limit_bytes=100 * 1024 * 1024,
        ),
    )(x, w_mat)

    del amax
    return y


# device time: 378228 ns/iter; 1.1475x vs baseline; 1.0772x over previous
import functools

import jax
import jax.numpy as jnp
from jax import lax
from jax.experimental import pallas as pl
from jax.experimental.pallas import tpu as pltpu

N_DEV = 32
N_STREAM = 4


def kernel(x, w_mat):
    m_glob, k_loc = x.shape
    k2, n = w_mat.shape
    assert k_loc == k2
    blk = m_glob // N_DEV
    nh = n // 2
    ns = nh // 2

    def body(x_ref, w_ref, out_ref, amax_ref,
             slots, send_sems, recv_sems,
             bf_send, bf_recv, bf_send_sems, bf_recv_sems):
        my = lax.axis_index("i")
        left = lax.rem(my - 1 + N_DEV, N_DEV)
        right = lax.rem(my + 1, N_DEV)

        barrier_sem = pltpu.get_barrier_semaphore()
        for nbr in (left, right):
            pl.semaphore_signal(
                barrier_sem, inc=1,
                device_id=(nbr,), device_id_type=pl.DeviceIdType.MESH,
            )
        pl.semaphore_wait(barrier_sem, 2)

        def gemms(h):
            o_r = lax.rem(my - 1 - h + 2 * N_DEV, N_DEV)
            o_l = lax.rem(my + 1 + h, N_DEV)
            p_r = jnp.dot(
                x_ref[pl.ds(o_r * blk, blk), :], w_ref[:, pl.ds(0, nh)],
                preferred_element_type=jnp.float32,
                precision=lax.Precision.HIGHEST,
            )
            p_l = jnp.dot(
                x_ref[pl.ds(o_l * blk, blk), :], w_ref[:, pl.ds(nh, nh)],
                preferred_element_type=jnp.float32,
                precision=lax.Precision.HIGHEST,
            )
            return p_r, p_l

        streams = [(right, 0), (left, nh), (right, ns), (left, nh + ns)]

        send_rdmas = []
        p_r, p_l = gemms(0)
        for h in range(N_DEV):
            parts = (p_r[:, :ns], p_l[:, :ns], p_r[:, ns:], p_l[:, ns:])
            for s, ((dst, col), part) in enumerate(zip(streams, parts)):
                if h == 0:
                    slots[s, h] = part
                else:
                    recv = pltpu.make_async_remote_copy(
                        src_ref=slots.at[s, h], dst_ref=slots.at[s, h],
                        send_sem=send_sems.at[s, h],
                        recv_sem=recv_sems.at[s, h],
                        device_id=(dst,),
                        device_id_type=pl.DeviceIdType.MESH,
                    )
                    recv.wait_recv()
                    if h == N_DEV - 1:
                        out_ref[:, pl.ds(col, ns)] = slots[s, h] + part
                        continue
                    slots[s, h] = slots[s, h] + part
                rdma = pltpu.make_async_remote_copy(
                    src_ref=slots.at[s, h], dst_ref=slots.at[s, h + 1],
                    send_sem=send_sems.at[s, h],
                    recv_sem=recv_sems.at[s, h + 1],
                    device_id=(dst,), device_id_type=pl.DeviceIdType.MESH,
                )
                rdma.start()
                send_rdmas.append(rdma)
            if h < N_DEV - 1:
                p_r, p_l = gemms(h + 1)

        amax = jnp.max(jnp.abs(out_ref[...]))
        for k in range(5):
            partner = my ^ (1 << k)
            bf_send[k] = jnp.broadcast_to(amax, (8, 128))
            ex = pltpu.make_async_remote_copy(
                src_ref=bf_send.at[k], dst_ref=bf_recv.at[k],
                send_sem=bf_send_sems.at[k], recv_sem=bf_recv_sems.at[k],
                device_id=(partner,), device_id_type=pl.DeviceIdType.MESH,
            )
            ex.start()
            ex.wait()
            amax = jnp.maximum(amax, bf_recv[k][0, 0])
        amax_ref[...] = jnp.broadcast_to(amax, (8, 128))

        scale = amax / 448.0
        v = jnp.clip(out_ref[...] / scale, -448.0, 448.0)
        q = v.astype(jnp.float8_e4m3fn).astype(jnp.float32)
        out_ref[...] = q * scale

        for rdma in send_rdmas:
            rdma.wait_send()

        @functools.partial(
            pl.run_scoped, second_barrier=pltpu.SemaphoreType.REGULAR
        )
        def _(second_barrier):
            for nbr in (left, right):
                pl.semaphore_signal(
                    second_barrier, inc=1,
                    device_id=(nbr,), device_id_type=pl.DeviceIdType.MESH,
                )
            pl.semaphore_wait(second_barrier, 2)

    y, amax = pl.pallas_call(
        body,
        out_shape=[
            jax.ShapeDtypeStruct((blk, n), jnp.float32),
            jax.ShapeDtypeStruct((8, 128), jnp.float32),
        ],
        in_specs=[
            pl.BlockSpec(memory_space=pltpu.VMEM),
            pl.BlockSpec(memory_space=pltpu.VMEM),
        ],
        out_specs=[
            pl.BlockSpec(memory_space=pltpu.VMEM),
            pl.BlockSpec(memory_space=pltpu.VMEM),
        ],
        scratch_shapes=[
            pltpu.VMEM((N_STREAM, N_DEV, blk, ns), jnp.float32),
            pltpu.SemaphoreType.DMA((N_STREAM, N_DEV)),
            pltpu.SemaphoreType.DMA((N_STREAM, N_DEV)),
            pltpu.VMEM((5, 8, 128), jnp.float32),
            pltpu.VMEM((5, 8, 128), jnp.float32),
            pltpu.SemaphoreType.DMA((5,)),
            pltpu.SemaphoreType.DMA((5,)),
        ],
        compiler_params=pltpu.CompilerParams(
            collective_id=0,
            vmem_limit_bytes=100 * 1024 * 1024,
        ),
    )(x, w_mat)

    del amax
    return y


# device time: 207615 ns/iter; 2.0905x vs baseline; 1.8218x over previous
import functools

import jax
import jax.numpy as jnp
import numpy as np
from jax import lax
from jax.experimental import pallas as pl
from jax.experimental.pallas import tpu as pltpu

N_DEV = 32
N_STREAM = 4
LOG2_DEV = 5


def _ring_tables():
    plane = [(0, 0), (1, 0), (1, 1), (0, 1), (0, 2), (1, 2), (1, 3), (0, 3)]
    lid = {}
    for z in range(4):
        for i, (xx, yy) in enumerate(plane):
            lid[(xx, yy, z)] = z * 8 + i
    p_yz = [(0, 0), (1, 0), (2, 0), (3, 0), (3, 1), (2, 1), (1, 1), (0, 1),
            (0, 2), (1, 2), (2, 2), (3, 2), (3, 3), (2, 3), (1, 3), (0, 3)]
    cyc = [(0, y, z) for (y, z) in p_yz] + \
          [(1, y, z) for (y, z) in reversed(p_yz)]
    seq = np.array([lid[c] for c in cyc], dtype=np.int32)
    pos = np.empty(N_DEV, np.int32)
    succ = np.empty(N_DEV, np.int32)
    pred = np.empty(N_DEV, np.int32)
    for i, a in enumerate(seq):
        pos[a] = i
        b = seq[(i + 1) % N_DEV]
        succ[a] = b
        pred[b] = a
    own_f = np.empty((N_DEV, N_DEV), np.int32)
    own_b = np.empty((N_DEV, N_DEV), np.int32)
    cur_f = pred.copy()
    cur_b = succ.copy()
    for h in range(N_DEV):
        own_f[h] = cur_f
        own_b[h] = cur_b
        cur_f = pred[cur_f]
        cur_b = succ[cur_b]
    assert (own_f[N_DEV - 1] == np.arange(N_DEV)).all()
    assert (own_b[N_DEV - 1] == np.arange(N_DEV)).all()
    bf = np.empty((LOG2_DEV, N_DEV), np.int32)
    for k in range(LOG2_DEV):
        bf[k] = seq[pos ^ (1 << k)]
    return seq, succ, pred, own_f, own_b, bf


_SEQ, _SUCC, _PRED, _OWN_F, _OWN_B, _BF = _ring_tables()


def kernel(x, w_mat):
    m_glob, k_loc = x.shape
    k2, n = w_mat.shape
    assert k_loc == k2
    blk = m_glob // N_DEV
    nh = n // 2
    ns = nh // 2

    my = lax.axis_index("i")
    sched_f = jnp.asarray(_OWN_F)[:, my]
    sched_b = jnp.asarray(_OWN_B)[:, my]
    nbrs = jnp.concatenate([
        jnp.asarray(_SUCC)[my][None],
        jnp.asarray(_PRED)[my][None],
        jnp.asarray(_BF)[:, my],
    ]).astype(jnp.int32)

    def body(sched_f_ref, sched_b_ref, nbrs_ref, x_ref, w_ref,
             out_ref, amax_ref,
             slots, send_sems, recv_sems,
             bf_send, bf_recv, bf_send_sems, bf_recv_sems):
        fwd = nbrs_ref[0]
        bwd = nbrs_ref[1]

        barrier_sem = pltpu.get_barrier_semaphore()
        for nbr in (fwd, bwd):
            pl.semaphore_signal(
                barrier_sem, inc=1,
                device_id=(nbr,), device_id_type=pl.DeviceIdType.MESH,
            )
        pl.semaphore_wait(barrier_sem, 2)

        def gemms(h):
            o_f = sched_f_ref[h]
            o_b = sched_b_ref[h]
            p_f = jnp.dot(
                x_ref[pl.ds(o_f * blk, blk), :], w_ref[:, pl.ds(0, nh)],
                preferred_element_type=jnp.float32,
                precision=lax.Precision.HIGHEST,
            )
            p_b = jnp.dot(
                x_ref[pl.ds(o_b * blk, blk), :], w_ref[:, pl.ds(nh, nh)],
                preferred_element_type=jnp.float32,
                precision=lax.Precision.HIGHEST,
            )
            return p_f, p_b

        streams = [(fwd, 0), (bwd, nh), (fwd, ns), (bwd, nh + ns)]

        send_rdmas = []
        p_f, p_b = gemms(0)
        for h in range(N_DEV):
            parts = (p_f[:, :ns], p_b[:, :ns], p_f[:, ns:], p_b[:, ns:])
            for s, ((dst, col), part) in enumerate(zip(streams, parts)):
                if h == 0:
                    slots[s, h] = part
                else:
                    recv = pltpu.make_async_remote_copy(
                        src_ref=slots.at[s, h], dst_ref=slots.at[s, h],
                        send_sem=send_sems.at[s, h],
                        recv_sem=recv_sems.at[s, h],
                        device_id=(dst,),
                        device_id_type=pl.DeviceIdType.MESH,
                    )
                    recv.wait_recv()
                    if h == N_DEV - 1:
                        out_ref[:, pl.ds(col, ns)] = slots[s, h] + part
                        continue
                    slots[s, h] = slots[s, h] + part
                rdma = pltpu.make_async_remote_copy(
                    src_ref=slots.at[s, h], dst_ref=slots.at[s, h + 1],
                    send_sem=send_sems.at[s, h],
                    recv_sem=recv_sems.at[s, h + 1],
                    device_id=(dst,), device_id_type=pl.DeviceIdType.MESH,
                )
                rdma.start()
                send_rdmas.append(rdma)
            if h < N_DEV - 1:
                p_f, p_b = gemms(h + 1)

        amax = jnp.max(jnp.abs(out_ref[...]))
        for k in range(LOG2_DEV):
            partner = nbrs_ref[2 + k]
            bf_send[k] = jnp.broadcast_to(amax, (8, 128))
            ex = pltpu.make_async_remote_copy(
                src_ref=bf_send.at[k], dst_ref=bf_recv.at[k],
                send_sem=bf_send_sems.at[k], recv_sem=bf_recv_sems.at[k],
                device_id=(partner,), device_id_type=pl.DeviceIdType.MESH,
            )
            ex.start()
            ex.wait()
            amax = jnp.maximum(amax, bf_recv[k][0, 0])
        amax_ref[...] = jnp.broadcast_to(amax, (8, 128))

        scale = amax / 448.0
        v = jnp.clip(out_ref[...] / scale, -448.0, 448.0)
        q = v.astype(jnp.float8_e4m3fn).astype(jnp.float32)
        out_ref[...] = q * scale

        for rdma in send_rdmas:
            rdma.wait_send()

        @functools.partial(
            pl.run_scoped, second_barrier=pltpu.SemaphoreType.REGULAR
        )
        def _(second_barrier):
            for nbr in (fwd, bwd):
                pl.semaphore_signal(
                    second_barrier, inc=1,
                    device_id=(nbr,), device_id_type=pl.DeviceIdType.MESH,
                )
            pl.semaphore_wait(second_barrier, 2)

    y, amax = pl.pallas_call(
        body,
        out_shape=[
            jax.ShapeDtypeStruct((blk, n), jnp.float32),
            jax.ShapeDtypeStruct((8, 128), jnp.float32),
        ],
        in_specs=[
            pl.BlockSpec(memory_space=pltpu.SMEM),
            pl.BlockSpec(memory_space=pltpu.SMEM),
            pl.BlockSpec(memory_space=pltpu.SMEM),
            pl.BlockSpec(memory_space=pltpu.VMEM),
            pl.BlockSpec(memory_space=pltpu.VMEM),
        ],
        out_specs=[
            pl.BlockSpec(memory_space=pltpu.VMEM),
            pl.BlockSpec(memory_space=pltpu.VMEM),
        ],
        scratch_shapes=[
            pltpu.VMEM((N_STREAM, N_DEV, blk, ns), jnp.float32),
            pltpu.SemaphoreType.DMA((N_STREAM, N_DEV)),
            pltpu.SemaphoreType.DMA((N_STREAM, N_DEV)),
            pltpu.VMEM((LOG2_DEV, 8, 128), jnp.float32),
            pltpu.VMEM((LOG2_DEV, 8, 128), jnp.float32),
            pltpu.SemaphoreType.DMA((LOG2_DEV,)),
            pltpu.SemaphoreType.DMA((LOG2_DEV,)),
        ],
        compiler_params=pltpu.CompilerParams(
            collective_id=0,
            vmem_limit_bytes=100 * 1024 * 1024,
        ),
    )(sched_f, sched_b, nbrs, x, w_mat)

    del amax
    return y


# device time: 203645 ns/iter; 2.1313x vs baseline; 1.0195x over previous
import functools

import jax
import jax.numpy as jnp
import numpy as np
from jax import lax
from jax.experimental import pallas as pl
from jax.experimental.pallas import tpu as pltpu

N_DEV = 32
N_STREAM = 4
LOG2_DEV = 5


def _ring_tables():
    plane = [(0, 0), (1, 0), (1, 1), (0, 1), (0, 2), (1, 2), (1, 3), (0, 3)]
    lid = {}
    for z in range(4):
        for i, (xx, yy) in enumerate(plane):
            lid[(xx, yy, z)] = z * 8 + i
    p_yz = [(0, 0), (1, 0), (2, 0), (3, 0), (3, 1), (2, 1), (1, 1), (0, 1),
            (0, 2), (1, 2), (2, 2), (3, 2), (3, 3), (2, 3), (1, 3), (0, 3)]
    cyc = [(0, y, z) for (y, z) in p_yz] + \
          [(1, y, z) for (y, z) in reversed(p_yz)]
    seq = np.array([lid[c] for c in cyc], dtype=np.int32)
    pos = np.empty(N_DEV, np.int32)
    succ = np.empty(N_DEV, np.int32)
    pred = np.empty(N_DEV, np.int32)
    for i, a in enumerate(seq):
        pos[a] = i
        b = seq[(i + 1) % N_DEV]
        succ[a] = b
        pred[b] = a
    own_f = np.empty((N_DEV, N_DEV), np.int32)
    own_b = np.empty((N_DEV, N_DEV), np.int32)
    cur_f = pred.copy()
    cur_b = succ.copy()
    for h in range(N_DEV):
        own_f[h] = cur_f
        own_b[h] = cur_b
        cur_f = pred[cur_f]
        cur_b = succ[cur_b]
    assert (own_f[N_DEV - 1] == np.arange(N_DEV)).all()
    assert (own_b[N_DEV - 1] == np.arange(N_DEV)).all()
    bf = np.empty((LOG2_DEV, N_DEV), np.int32)
    for k in range(LOG2_DEV):
        bf[k] = seq[pos ^ (1 << k)]
    return seq, succ, pred, own_f, own_b, bf


_SEQ, _SUCC, _PRED, _OWN_F, _OWN_B, _BF = _ring_tables()


def kernel(x, w_mat):
    m_glob, k_loc = x.shape
    k2, n = w_mat.shape
    assert k_loc == k2
    blk = m_glob // N_DEV
    nh = n // 2
    ns = nh // 2

    my = lax.axis_index("i")
    sched_f = jnp.asarray(_OWN_F)[:, my]
    sched_b = jnp.asarray(_OWN_B)[:, my]
    nbrs = jnp.concatenate([
        jnp.asarray(_SUCC)[my][None],
        jnp.asarray(_PRED)[my][None],
        jnp.asarray(_BF)[:, my],
    ]).astype(jnp.int32)

    def body(sched_f_ref, sched_b_ref, nbrs_ref, x_ref, w_ref,
             out_ref, amax_ref,
             slots, send_sems, recv_sems,
             bf_send, bf_recv, bf_send_sems, bf_recv_sems):
        fwd = nbrs_ref[0]
        bwd = nbrs_ref[1]

        barrier_sem = pltpu.get_barrier_semaphore()
        for nbr in (fwd, bwd):
            pl.semaphore_signal(
                barrier_sem, inc=1,
                device_id=(nbr,), device_id_type=pl.DeviceIdType.MESH,
            )
        pl.semaphore_wait(barrier_sem, 2)

        def gemms(h):
            o_f = sched_f_ref[h]
            o_b = sched_b_ref[h]
            p_f = jnp.dot(
                x_ref[pl.ds(o_f * blk, blk), :], w_ref[:, pl.ds(0, nh)],
                preferred_element_type=jnp.float32,
                precision=lax.Precision.HIGHEST,
            )
            p_b = jnp.dot(
                x_ref[pl.ds(o_b * blk, blk), :], w_ref[:, pl.ds(nh, nh)],
                preferred_element_type=jnp.float32,
                precision=lax.Precision.HIGHEST,
            )
            return p_f, p_b

        streams = [(fwd, 0), (bwd, nh), (fwd, ns), (bwd, nh + ns)]

        send_rdmas = []
        p_f, p_b = gemms(0)
        for h in range(N_DEV):
            parts = (p_f[:, :ns], p_b[:, :ns], p_f[:, ns:], p_b[:, ns:])
            for s, ((dst, col), part) in enumerate(zip(streams, parts)):
                if h == 0:
                    slots[s, h] = part
                else:
                    recv = pltpu.make_async_remote_copy(
                        src_ref=slots.at[s, h], dst_ref=slots.at[s, h],
                        send_sem=send_sems.at[s, h],
                        recv_sem=recv_sems.at[s, h],
                        device_id=(dst,),
                        device_id_type=pl.DeviceIdType.MESH,
                    )
                    recv.wait_recv()
                    if h == N_DEV - 1:
                        out_ref[:, pl.ds(col, ns)] = slots[s, h] + part
                        continue
                    slots[s, h] = slots[s, h] + part
                rdma = pltpu.make_async_remote_copy(
                    src_ref=slots.at[s, h], dst_ref=slots.at[s, h + 1],
                    send_sem=send_sems.at[s, h],
                    recv_sem=recv_sems.at[s, h + 1],
                    device_id=(dst,), device_id_type=pl.DeviceIdType.MESH,
                )
                rdma.start()
                send_rdmas.append(rdma)
            if h < N_DEV - 1:
                p_f, p_b = gemms(h + 1)

        my_id = lax.axis_index("i")
        amax = jnp.max(jnp.abs(out_ref[...]))
        bf_send[0] = jnp.broadcast_to(amax, (8, 128))
        bf_rdmas = []
        for j in range(1, N_DEV):
            tgt = lax.rem(my_id + j, N_DEV)
            ex = pltpu.make_async_remote_copy(
                src_ref=bf_send.at[0], dst_ref=bf_recv.at[j],
                send_sem=bf_send_sems.at[j], recv_sem=bf_recv_sems.at[j],
                device_id=(tgt,), device_id_type=pl.DeviceIdType.MESH,
            )
            ex.start()
            bf_rdmas.append(ex)
        for j in range(1, N_DEV):
            bf_rdmas[j - 1].wait_recv()
            amax = jnp.maximum(amax, bf_recv[j][0, 0])
        amax_ref[...] = jnp.broadcast_to(amax, (8, 128))

        scale = amax / 448.0
        v = jnp.clip(out_ref[...] / scale, -448.0, 448.0)
        q = v.astype(jnp.float8_e4m3fn).astype(jnp.float32)
        out_ref[...] = q * scale

        for rdma in send_rdmas:
            rdma.wait_send()
        for rdma in bf_rdmas:
            rdma.wait_send()

        @functools.partial(
            pl.run_scoped, second_barrier=pltpu.SemaphoreType.REGULAR
        )
        def _(second_barrier):
            for nbr in (fwd, bwd):
                pl.semaphore_signal(
                    second_barrier, inc=1,
                    device_id=(nbr,), device_id_type=pl.DeviceIdType.MESH,
                )
            pl.semaphore_wait(second_barrier, 2)

    y, amax = pl.pallas_call(
        body,
        out_shape=[
            jax.ShapeDtypeStruct((blk, n), jnp.float32),
            jax.ShapeDtypeStruct((8, 128), jnp.float32),
        ],
        in_specs=[
            pl.BlockSpec(memory_space=pltpu.SMEM),
            pl.BlockSpec(memory_space=pltpu.SMEM),
            pl.BlockSpec(memory_space=pltpu.SMEM),
            pl.BlockSpec(memory_space=pltpu.VMEM),
            pl.BlockSpec(memory_space=pltpu.VMEM),
        ],
        out_specs=[
            pl.BlockSpec(memory_space=pltpu.VMEM),
            pl.BlockSpec(memory_space=pltpu.VMEM),
        ],
        scratch_shapes=[
            pltpu.VMEM((N_STREAM, N_DEV, blk, ns), jnp.float32),
            pltpu.SemaphoreType.DMA((N_STREAM, N_DEV)),
            pltpu.SemaphoreType.DMA((N_STREAM, N_DEV)),
            pltpu.VMEM((1, 8, 128), jnp.float32),
            pltpu.VMEM((N_DEV, 8, 128), jnp.float32),
            pltpu.SemaphoreType.DMA((N_DEV,)),
            pltpu.SemaphoreType.DMA((N_DEV,)),
        ],
        compiler_params=pltpu.CompilerParams(
            collective_id=0,
            vmem_limit_bytes=100 * 1024 * 1024,
        ),
    )(sched_f, sched_b, nbrs, x, w_mat)

    del amax
    return y


# device time: 202864 ns/iter; 2.1395x vs baseline; 1.0038x over previous
import functools

import jax
import jax.numpy as jnp
import numpy as np
from jax import lax
from jax.experimental import pallas as pl
from jax.experimental.pallas import tpu as pltpu

N_DEV = 32
N_STREAM = 4
LOG2_DEV = 5


def _ring_tables():
    plane = [(0, 0), (1, 0), (1, 1), (0, 1), (0, 2), (1, 2), (1, 3), (0, 3)]
    lid = {}
    for z in range(4):
        for i, (xx, yy) in enumerate(plane):
            lid[(xx, yy, z)] = z * 8 + i
    p_yz = [(0, 0), (1, 0), (2, 0), (3, 0), (3, 1), (2, 1), (1, 1), (0, 1),
            (0, 2), (1, 2), (2, 2), (3, 2), (3, 3), (2, 3), (1, 3), (0, 3)]
    cyc = [(0, y, z) for (y, z) in p_yz] + \
          [(1, y, z) for (y, z) in reversed(p_yz)]
    seq = np.array([lid[c] for c in cyc], dtype=np.int32)
    pos = np.empty(N_DEV, np.int32)
    succ = np.empty(N_DEV, np.int32)
    pred = np.empty(N_DEV, np.int32)
    for i, a in enumerate(seq):
        pos[a] = i
        b = seq[(i + 1) % N_DEV]
        succ[a] = b
        pred[b] = a
    own_f = np.empty((N_DEV, N_DEV), np.int32)
    own_b = np.empty((N_DEV, N_DEV), np.int32)
    cur_f = pred.copy()
    cur_b = succ.copy()
    for h in range(N_DEV):
        own_f[h] = cur_f
        own_b[h] = cur_b
        cur_f = pred[cur_f]
        cur_b = succ[cur_b]
    assert (own_f[N_DEV - 1] == np.arange(N_DEV)).all()
    assert (own_b[N_DEV - 1] == np.arange(N_DEV)).all()
    bf = np.empty((LOG2_DEV, N_DEV), np.int32)
    for k in range(LOG2_DEV):
        bf[k] = seq[pos ^ (1 << k)]
    return seq, succ, pred, own_f, own_b, bf


_SEQ, _SUCC, _PRED, _OWN_F, _OWN_B, _BF = _ring_tables()


def kernel(x, w_mat):
    m_glob, k_loc = x.shape
    k2, n = w_mat.shape
    assert k_loc == k2
    blk = m_glob // N_DEV
    nh = n // 2
    ns = nh // 2

    my = lax.axis_index("i")
    sched_f = jnp.asarray(_OWN_F)[:, my]
    sched_b = jnp.asarray(_OWN_B)[:, my]
    nbrs = jnp.concatenate([
        jnp.asarray(_SUCC)[my][None],
        jnp.asarray(_PRED)[my][None],
        jnp.asarray(_BF)[:, my],
    ]).astype(jnp.int32)

    def body(sched_f_ref, sched_b_ref, nbrs_ref, x_ref, w_ref,
             out_ref, amax_ref,
             slots, send_sems, recv_sems,
             bf_send, bf_recv, bf_send_sems, bf_recv_sems):
        fwd = nbrs_ref[0]
        bwd = nbrs_ref[1]

        def gemms(h):
            o_f = sched_f_ref[h]
            o_b = sched_b_ref[h]
            p_f = jnp.dot(
                x_ref[pl.ds(o_f * blk, blk), :], w_ref[:, pl.ds(0, nh)],
                preferred_element_type=jnp.float32,
                precision=lax.Precision.HIGHEST,
            )
            p_b = jnp.dot(
                x_ref[pl.ds(o_b * blk, blk), :], w_ref[:, pl.ds(nh, nh)],
                preferred_element_type=jnp.float32,
                precision=lax.Precision.HIGHEST,
            )
            return p_f, p_b

        streams = [(fwd, 0), (bwd, nh), (fwd, ns), (bwd, nh + ns)]

        send_rdmas = []
        barrier_sem = pltpu.get_barrier_semaphore()
        for nbr in (fwd, bwd):
            pl.semaphore_signal(
                barrier_sem, inc=1,
                device_id=(nbr,), device_id_type=pl.DeviceIdType.MESH,
            )
        p_f, p_b = gemms(0)
        pl.semaphore_wait(barrier_sem, 2)
        for h in range(N_DEV):
            parts = (p_f[:, :ns], p_b[:, :ns], p_f[:, ns:], p_b[:, ns:])
            for s, ((dst, col), part) in enumerate(zip(streams, parts)):
                if h == 0:
                    slots[s, h] = part
                else:
                    recv = pltpu.make_async_remote_copy(
                        src_ref=slots.at[s, h], dst_ref=slots.at[s, h],
                        send_sem=send_sems.at[s, h],
                        recv_sem=recv_sems.at[s, h],
                        device_id=(dst,),
                        device_id_type=pl.DeviceIdType.MESH,
                    )
                    recv.wait_recv()
                    if h == N_DEV - 1:
                        out_ref[:, pl.ds(col, ns)] = slots[s, h] + part
                        continue
                    slots[s, h] = slots[s, h] + part
                rdma = pltpu.make_async_remote_copy(
                    src_ref=slots.at[s, h], dst_ref=slots.at[s, h + 1],
                    send_sem=send_sems.at[s, h],
                    recv_sem=recv_sems.at[s, h + 1],
                    device_id=(dst,), device_id_type=pl.DeviceIdType.MESH,
                )
                rdma.start()
                send_rdmas.append(rdma)
            if h < N_DEV - 1:
                p_f, p_b = gemms(h + 1)

        my_id = lax.axis_index("i")
        amax = jnp.max(jnp.abs(out_ref[...]))
        bf_send[0] = jnp.broadcast_to(amax, (8, 128))
        bf_rdmas = []
        for j in range(1, N_DEV):
            tgt = lax.rem(my_id + j, N_DEV)
            ex = pltpu.make_async_remote_copy(
                src_ref=bf_send.at[0], dst_ref=bf_recv.at[j],
                send_sem=bf_send_sems.at[j], recv_sem=bf_recv_sems.at[j],
                device_id=(tgt,), device_id_type=pl.DeviceIdType.MESH,
            )
            ex.start()
            bf_rdmas.append(ex)
        for j in range(1, N_DEV):
            bf_rdmas[j - 1].wait_recv()
            amax = jnp.maximum(amax, bf_recv[j][0, 0])
        amax_ref[...] = jnp.broadcast_to(amax, (8, 128))

        scale = amax / 448.0
        inv_scale = 448.0 / amax
        q = (out_ref[...] * inv_scale).astype(jnp.float8_e4m3fn)
        out_ref[...] = q.astype(jnp.float32) * scale

        for rdma in send_rdmas:
            rdma.wait_send()
        for rdma in bf_rdmas:
            rdma.wait_send()

        @functools.partial(
            pl.run_scoped, second_barrier=pltpu.SemaphoreType.REGULAR
        )
        def _(second_barrier):
            for nbr in (fwd, bwd):
                pl.semaphore_signal(
                    second_barrier, inc=1,
                    device_id=(nbr,), device_id_type=pl.DeviceIdType.MESH,
                )
            pl.semaphore_wait(second_barrier, 2)

    y, amax = pl.pallas_call(
        body,
        out_shape=[
            jax.ShapeDtypeStruct((blk, n), jnp.float32),
            jax.ShapeDtypeStruct((8, 128), jnp.float32),
        ],
        in_specs=[
            pl.BlockSpec(memory_space=pltpu.SMEM),
            pl.BlockSpec(memory_space=pltpu.SMEM),
            pl.BlockSpec(memory_space=pltpu.SMEM),
            pl.BlockSpec(memory_space=pltpu.VMEM),
            pl.BlockSpec(memory_space=pltpu.VMEM),
        ],
        out_specs=[
            pl.BlockSpec(memory_space=pltpu.VMEM),
            pl.BlockSpec(memory_space=pltpu.VMEM),
        ],
        scratch_shapes=[
            pltpu.VMEM((N_STREAM, N_DEV, blk, ns), jnp.float32),
            pltpu.SemaphoreType.DMA((N_STREAM, N_DEV)),
            pltpu.SemaphoreType.DMA((N_STREAM, N_DEV)),
            pltpu.VMEM((1, 8, 128), jnp.float32),
            pltpu.VMEM((N_DEV, 8, 128), jnp.float32),
            pltpu.SemaphoreType.DMA((N_DEV,)),
            pltpu.SemaphoreType.DMA((N_DEV,)),
        ],
        compiler_params=pltpu.CompilerParams(
            collective_id=0,
            vmem_limit_bytes=100 * 1024 * 1024,
        ),
    )(sched_f, sched_b, nbrs, x, w_mat)

    del amax
    return y
